# baseline (speedup 1.0000x reference)
import functools

import jax
import jax.numpy as jnp
from jax.experimental import pallas as pl
from jax.experimental.pallas import tpu as pltpu

_HIGH = jax.lax.Precision.HIGHEST


def _silu(v):
    return v * jax.nn.sigmoid(v)


def _dot(a, b):
    return jax.lax.dot_general(a, b, (((1,), (0,)), ((), ())),
                               preferred_element_type=jnp.float32,
                               precision=_HIGH)


def _bd(w, k):
    a, b = w.shape
    out = jnp.zeros((k * a, k * b), jnp.float32)
    for i in range(k):
        out = out.at[i * a:(i + 1) * a, i * b:(i + 1) * b].set(w)
    return out


def _edge_body(guv_ref, diff_ref, w1c_ref, w2_ref, w3_ref, w4_ref,
               sw1_ref, sw2_ref, sw3_ref, dp1_ref, dp2_ref, sel_ref,
               rep3_ref, b1_ref, b2_ref, b3_ref, b4_ref, sb1_ref, sb2_ref,
               dpb1_ref, dpb2_ref, m_ref, pos_ref):
    diff = diff_ref[...]
    sq = _dot(diff * diff, sel_ref[...])
    dist = jnp.sqrt(sq + 1e-10)
    d1 = _silu(_dot(dist, dp1_ref[...]) + dpb1_ref[...])
    d = _dot(d1, dp2_ref[...]) + dpb2_ref[...]
    m1 = guv_ref[...] + _dot(d, w1c_ref[...]) + b1_ref[...]
    m2 = _dot(_silu(m1), w2_ref[...]) + b2_ref[...]
    m3 = _dot(_silu(m2), w3_ref[...]) + b3_ref[...]
    m4 = _dot(_silu(m3), w4_ref[...]) + b4_ref[...]
    m_ref[...] = m4
    a1 = _silu(_dot(m4, sw1_ref[...]) + sb1_ref[...])
    a2 = _silu(_dot(a1, sw2_ref[...]) + sb2_ref[...])
    sw = jnp.tanh(_dot(a2, sw3_ref[...]))
    pos_ref[...] = diff * _dot(sw, rep3_ref[...])


@functools.partial(jax.jit, static_argnums=())
def _edge_layer(guv4, diff4, wd):
    r_total = guv4.shape[0]
    r_blk = 2000
    if r_total % r_blk != 0:
        r_blk = 8
        pad = (-r_total) % r_blk
        if pad:
            guv4 = jnp.pad(guv4, ((0, pad), (0, 0)))
            diff4 = jnp.pad(diff4, ((0, pad), (0, 0)))
    r_pad = guv4.shape[0]
    grid = r_pad // r_blk

    def espec(cols):
        return pl.BlockSpec((r_blk, cols), lambda i: (i, 0))

    def wspec(shape):
        return pl.BlockSpec(shape, lambda i: (0, 0))

    in_specs = [espec(256), espec(12)]
    weights = [wd['w1c'], wd['w2'], wd['w3'], wd['w4'], wd['sw1'], wd['sw2'],
               wd['sw3'], wd['dp1'], wd['dp2'], wd['sel'], wd['rep3'],
               wd['b1'], wd['b2'], wd['b3'], wd['b4'], wd['sb1'], wd['sb2'],
               wd['dpb1'], wd['dpb2']]
    in_specs += [wspec(w.shape) for w in weights]

    m4, pos4 = pl.pallas_call(
        _edge_body,
        grid=(grid,),
        in_specs=in_specs,
        out_specs=[espec(256), espec(12)],
        out_shape=[jax.ShapeDtypeStruct((r_pad, 256), jnp.float32),
                   jax.ShapeDtypeStruct((r_pad, 12), jnp.float32)],
    )(guv4, diff4, *weights)
    return m4[:r_total], pos4[:r_total]


def kernel(x, h, t, edge_index, params):
    p = params
    n = x.shape[0]
    e = edge_index.shape[1]
    te = p['tp_w1'].shape[0]
    row = edge_index[0]
    col = edge_index[1]

    i = jnp.arange(te // 2)
    freq = 10000.0 ** (2.0 * i / te)
    tt = t.reshape(-1, 1)
    temb = jnp.concatenate([jnp.sin(tt / freq), jnp.cos(tt / freq)], axis=1)
    tp = jnp.dot(_silu(jnp.dot(temb, p['tp_w1'], precision=_HIGH) + p['tp_b1']),
                 p['tp_w2'], precision=_HIGH) + p['tp_b2']
    hh = jnp.dot(_silu(jnp.dot(h, p['hp_w1'], precision=_HIGH) + p['hp_b1']),
                 p['hp_w2'], precision=_HIGH) + p['hp_b2']
    xx = x

    sel = jnp.zeros((12, 4), jnp.float32)
    for k in range(4):
        sel = sel.at[3 * k:3 * k + 3, k].set(1.0)
    rep3 = jnp.zeros((4, 12), jnp.float32)
    for k in range(4):
        rep3 = rep3.at[k, 3 * k:3 * k + 3].set(1.0)
    shared = {
        'sw1': _bd(p['sw_w1'], 4), 'sw2': _bd(p['sw_w2'], 4),
        'sw3': _bd(p['sw_w3'], 4),
        'dp1': _bd(p['dp_w1'], 4), 'dp2': _bd(p['dp_w2'], 4),
        'sel': sel, 'rep3': rep3,
        'sb1': jnp.tile(p['sw_b1'], 4)[None, :],
        'sb2': jnp.tile(p['sw_b2'], 4)[None, :],
        'dpb1': jnp.tile(p['dp_b1'], 4)[None, :],
        'dpb2': jnp.tile(p['dp_b2'], 4)[None, :],
    }

    num_layers = len(p['msg'])
    for r in range(num_layers):
        mp = p['msg'][r]
        wd = dict(shared)
        wd['w1c'] = _bd(mp['w1'][128:160], 4)
        wd['w2'] = _bd(mp['w2'], 4)
        wd['w3'] = _bd(mp['w3'], 4)
        wd['w4'] = _bd(mp['w4'], 4)
        wd['b1'] = jnp.tile(mp['b1'], 4)[None, :]
        wd['b2'] = jnp.tile(mp['b2'], 4)[None, :]
        wd['b3'] = jnp.tile(mp['b3'], 4)[None, :]
        wd['b4'] = jnp.tile(mp['b4'], 4)[None, :]

        s = hh + tp
        u = jnp.dot(s, mp['w1'][:64], precision=_HIGH)
        v = jnp.dot(s, mp['w1'][64:128], precision=_HIGH)
        guv = jnp.take(u, row, axis=0) + jnp.take(v, col, axis=0)
        diff = jnp.take(xx, row, axis=0) - jnp.take(xx, col, axis=0)

        guv4 = guv.reshape(e // 4, 256)
        diff4 = diff.reshape(e // 4, 12)
        m4, pos4 = _edge_layer(guv4, diff4, wd)
        m = m4.reshape(e, 64)
        pos = pos4.reshape(e, 3)

        xx = xx + jax.ops.segment_sum(pos, row, num_segments=n)
        hh = hh + jax.ops.segment_sum(m, row, num_segments=n)

    hout = jnp.dot(hh, p['out_w'], precision=_HIGH) + p['out_b']
    return (xx, hout)
